# baseline (speedup 1.0000x reference)
import functools

import jax
import jax.numpy as jnp
from jax import lax
from jax.experimental import pallas as pl
from jax.experimental.pallas import tpu as pltpu
from jax.experimental.pallas import tpu_sc as plsc

_B, _T, _D = 16, 512, 256
_MAX_LEN = 2048
_DH = _D // 2
_CHUNK = 128
_NCHUNK = _MAX_LEN // _CHUNK
_GROUPS = _CHUNK // 16


def _idx_body(dur_ref, idx_ref, len_ref, cum_ref):
    dur = dur_ref[...]
    ii = lax.broadcasted_iota(jnp.int32, (_T, _T), 0)
    jj = lax.broadcasted_iota(jnp.int32, (_T, _T), 1)
    tri = (ii <= jj).astype(jnp.float32)
    cum = jnp.dot(dur.astype(jnp.float32), tri,
                  preferred_element_type=jnp.float32).astype(jnp.int32)
    total = cum[:, _T - 1]
    cum_ref[...] = cum
    pos2d = lax.broadcasted_iota(jnp.int32, (_MAX_LEN, _T), 0)
    ones_m = jnp.ones((_T,), jnp.float32)

    def body(b, carry):
        cum_b = cum_ref[pl.ds(b, 1), :]
        a = jnp.broadcast_to(cum_b, (_MAX_LEN, _T)) <= pos2d
        af = a.astype(jnp.float32)
        cnt = lax.dot_general(af, ones_m, (((1,), (0,)), ((), ())),
                              preferred_element_type=jnp.float32)
        idx_ref[pl.ds(b, 1), :] = jnp.minimum(cnt.astype(jnp.int32),
                                              _T)[None, :]
        return carry

    lax.fori_loop(0, _B, body, 0)
    len_ref[...] = jnp.broadcast_to(total[:, None], (_B, 128))


_idx_call = pl.pallas_call(
    _idx_body,
    out_shape=(
        jax.ShapeDtypeStruct((_B, _MAX_LEN), jnp.int32),
        jax.ShapeDtypeStruct((_B, 128), jnp.int32),
    ),
    scratch_shapes=[pltpu.VMEM((_B, _T), jnp.int32)],
)


_sc_mesh = plsc.VectorSubcoreMesh(core_axis_name="c", subcore_axis_name="s")


@functools.partial(
    pl.kernel,
    mesh=_sc_mesh,
    compiler_params=pltpu.CompilerParams(needs_layout_passes=False),
    out_type=jax.ShapeDtypeStruct((_B * _MAX_LEN, _D), jnp.float32),
    scratch_types=[
        pltpu.VMEM((_T + 8, _DH), jnp.float32),
        pltpu.VMEM((_MAX_LEN,), jnp.int32),
        pltpu.VMEM((_CHUNK, _DH), jnp.float32),
        pltpu.VMEM((_CHUNK, _DH), jnp.float32),
        pltpu.SemaphoreType.DMA,
        pltpu.SemaphoreType.DMA,
    ],
)
def _expand_call(x_hbm, z_hbm, idx_hbm, out_hbm, src_v, idx_v, buf_a, buf_b,
                 sem_a, sem_b):
    w = lax.axis_index("c") * 16 + lax.axis_index("s")
    b = w // 2
    dh = w % 2
    col0 = dh * _DH
    base = b * _MAX_LEN
    iota16 = lax.broadcasted_iota(jnp.int32, (16,), 0)
    cols = [iota16 + (16 * k) for k in range(_DH // 16)]

    pltpu.sync_copy(x_hbm.at[b, :, pl.ds(col0, _DH)], src_v.at[pl.ds(0, _T)])
    pltpu.sync_copy(z_hbm.at[:, pl.ds(col0, _DH)], src_v.at[pl.ds(_T, 8)])
    pltpu.sync_copy(idx_hbm.at[b], idx_v)

    def fill(buf, ch):
        for g in range(_GROUPS):
            rows = idx_v[pl.ds(ch * _CHUNK + g * 16, 16)]
            for j2 in range(0, 16, 2):
                s0 = jnp.full((16,), rows[j2], jnp.int32)
                s1 = jnp.full((16,), rows[j2 + 1], jnp.int32)
                vs0 = [plsc.load_gather(src_v, [s0, cols[k]])
                       for k in range(_DH // 16)]
                vs1 = [plsc.load_gather(src_v, [s1, cols[k]])
                       for k in range(_DH // 16)]
                for k in range(_DH // 16):
                    buf[g * 16 + j2, pl.ds(16 * k, 16)] = vs0[k]
                for k in range(_DH // 16):
                    buf[g * 16 + j2 + 1, pl.ds(16 * k, 16)] = vs1[k]

    def wr(buf, sem, ch):
        return pltpu.async_copy(
            buf, out_hbm.at[pl.ds(base + ch * _CHUNK, _CHUNK),
                            pl.ds(col0, _DH)], sem)

    def drain(buf, sem):
        pltpu.make_async_copy(
            buf, out_hbm.at[pl.ds(base, _CHUNK), pl.ds(col0, _DH)],
            sem).wait()

    def body(k, carry):
        ch0 = 2 * k

        @pl.when(k > 0)
        def _():
            drain(buf_a, sem_a)

        fill(buf_a, ch0)
        wr(buf_a, sem_a, ch0)

        @pl.when(k > 0)
        def _():
            drain(buf_b, sem_b)

        fill(buf_b, ch0 + 1)
        wr(buf_b, sem_b, ch0 + 1)
        return carry

    lax.fori_loop(0, _NCHUNK // 2, body, 0)
    drain(buf_a, sem_a)
    drain(buf_b, sem_b)


def kernel(x, duration, max_len):
    del max_len
    idx, mel = _idx_call(duration)
    zeros = jnp.zeros((8, _D), jnp.float32)
    out_flat = _expand_call(x, zeros, idx)
    return out_flat.reshape(_B, _MAX_LEN, _D), mel[:, 0]


# speedup vs baseline: 1.1181x; 1.1181x over previous
import functools

import jax
import jax.numpy as jnp
from jax import lax
from jax.experimental import pallas as pl
from jax.experimental.pallas import tpu as pltpu
from jax.experimental.pallas import tpu_sc as plsc

_B, _T, _D = 16, 512, 256
_MAX_LEN = 2048
_DH = _D // 2
_CHUNK = 64
_NCHUNK = _MAX_LEN // _CHUNK
_GROUPS = _CHUNK // 16


def _idx_body(dur_ref, idx_ref, len_ref, cum_ref):
    dur = dur_ref[...]
    ii = lax.broadcasted_iota(jnp.int32, (_T, _T), 0)
    jj = lax.broadcasted_iota(jnp.int32, (_T, _T), 1)
    tri = (ii <= jj).astype(jnp.float32)
    cum = jnp.dot(dur.astype(jnp.float32), tri,
                  preferred_element_type=jnp.float32).astype(jnp.int32)
    total = cum[:, _T - 1]
    cum_ref[...] = cum
    pos2d = lax.broadcasted_iota(jnp.int32, (_MAX_LEN, _T), 0)
    ones_m = jnp.ones((_T,), jnp.float32)

    def body(b, carry):
        cum_b = cum_ref[pl.ds(b, 1), :]
        a = jnp.broadcast_to(cum_b, (_MAX_LEN, _T)) <= pos2d
        af = a.astype(jnp.float32)
        cnt = lax.dot_general(af, ones_m, (((1,), (0,)), ((), ())),
                              preferred_element_type=jnp.float32)
        idx_ref[pl.ds(b, 1), :] = jnp.minimum(cnt.astype(jnp.int32),
                                              _T)[None, :]
        return carry

    lax.fori_loop(0, _B, body, 0)
    len_ref[...] = jnp.broadcast_to(total[:, None], (_B, 128))


_idx_call = pl.pallas_call(
    _idx_body,
    out_shape=(
        jax.ShapeDtypeStruct((_B, _MAX_LEN), jnp.int32),
        jax.ShapeDtypeStruct((_B, 128), jnp.int32),
    ),
    scratch_shapes=[pltpu.VMEM((_B, _T), jnp.int32)],
)


_sc_mesh = plsc.VectorSubcoreMesh(core_axis_name="c", subcore_axis_name="s")


@functools.partial(
    pl.kernel,
    mesh=_sc_mesh,
    compiler_params=pltpu.CompilerParams(needs_layout_passes=False),
    out_type=jax.ShapeDtypeStruct((_B * _MAX_LEN, _D), jnp.float32),
    scratch_types=[
        pltpu.VMEM((_T + 8, _DH), jnp.float32),
        pltpu.VMEM((_MAX_LEN,), jnp.int32),
        pltpu.VMEM((_CHUNK, _DH), jnp.float32),
        pltpu.VMEM((_CHUNK, _DH), jnp.float32),
        pltpu.SemaphoreType.DMA,
        pltpu.SemaphoreType.DMA,
    ],
)
def _expand_call(x_hbm, z_hbm, idx_hbm, out_hbm, src_v, idx_v, buf_a, buf_b,
                 sem_a, sem_b):
    w = lax.axis_index("c") * 16 + lax.axis_index("s")
    b = w // 2
    dh = w % 2
    col0 = dh * _DH
    base = b * _MAX_LEN
    iota16 = lax.broadcasted_iota(jnp.int32, (16,), 0)
    cols = [iota16 + (16 * k) for k in range(_DH // 16)]

    pltpu.sync_copy(x_hbm.at[b, :, pl.ds(col0, _DH)], src_v.at[pl.ds(0, _T)])
    pltpu.sync_copy(z_hbm.at[:, pl.ds(col0, _DH)], src_v.at[pl.ds(_T, 8)])
    pltpu.sync_copy(idx_hbm.at[b], idx_v)

    def fill(buf, ch):
        for g in range(_GROUPS):
            rows = idx_v[pl.ds(ch * _CHUNK + g * 16, 16)]
            for j2 in range(0, 16, 2):
                s0 = jnp.full((16,), rows[j2], jnp.int32)
                s1 = jnp.full((16,), rows[j2 + 1], jnp.int32)
                vs0 = [plsc.load_gather(src_v, [s0, cols[k]])
                       for k in range(_DH // 16)]
                vs1 = [plsc.load_gather(src_v, [s1, cols[k]])
                       for k in range(_DH // 16)]
                for k in range(_DH // 16):
                    buf[g * 16 + j2, pl.ds(16 * k, 16)] = vs0[k]
                for k in range(_DH // 16):
                    buf[g * 16 + j2 + 1, pl.ds(16 * k, 16)] = vs1[k]

    def wr(buf, sem, ch):
        return pltpu.async_copy(
            buf, out_hbm.at[pl.ds(base + ch * _CHUNK, _CHUNK),
                            pl.ds(col0, _DH)], sem)

    def drain(buf, sem):
        pltpu.make_async_copy(
            buf, out_hbm.at[pl.ds(base, _CHUNK), pl.ds(col0, _DH)],
            sem).wait()

    def body(k, carry):
        ch0 = 2 * k

        @pl.when(k > 0)
        def _():
            drain(buf_a, sem_a)

        fill(buf_a, ch0)
        wr(buf_a, sem_a, ch0)

        @pl.when(k > 0)
        def _():
            drain(buf_b, sem_b)

        fill(buf_b, ch0 + 1)
        wr(buf_b, sem_b, ch0 + 1)
        return carry

    lax.fori_loop(0, _NCHUNK // 2, body, 0)
    drain(buf_a, sem_a)
    drain(buf_b, sem_b)


def kernel(x, duration, max_len):
    del max_len
    idx, mel = _idx_call(duration)
    zeros = jnp.zeros((8, _D), jnp.float32)
    out_flat = _expand_call(x, zeros, idx)
    return out_flat.reshape(_B, _MAX_LEN, _D), mel[:, 0]
